# baseline (speedup 1.0000x reference)
import functools

import jax
import jax.numpy as jnp
from jax import lax
from jax.experimental import pallas as pl
from jax.experimental.pallas import tpu as pltpu
from jax.experimental.pallas import tpu_sc as plsc

_VOCAB = 1000000
_EMBED = 32
_NC = 2
_NS = 16
_NW = _NC * _NS
_CH = 128
_K = 5
_NG = 10
_NCH = _K * _NG
_BPW = _CH * _NCH
_B = _NW * _BPW
_NT = _B // _CH

_TC_C = 2048


@functools.partial(
    pl.pallas_call,
    out_shape=jax.ShapeDtypeStruct((_VOCAB, _EMBED), jnp.float32),
    grid=((_VOCAB + _TC_C - 1) // _TC_C,),
    in_specs=[pl.BlockSpec((_EMBED, _TC_C), lambda i: (0, i))],
    out_specs=pl.BlockSpec((_TC_C, _EMBED), lambda i: (i, 0)),
)
def _transpose_table(in_ref, out_ref):
    out_ref[...] = in_ref[...].T


@functools.partial(
    pl.kernel,
    mesh=plsc.VectorSubcoreMesh(core_axis_name="c", subcore_axis_name="s"),
    out_type=jax.ShapeDtypeStruct((_B, _EMBED), jnp.float32),
    scratch_types=[
        pltpu.VMEM((_NCH, _CH), jnp.int32),
        pltpu.VMEM((_K * _CH, _EMBED), jnp.float32),
        pltpu.VMEM((_K * _CH, _EMBED), jnp.float32),
        pltpu.VMEM((_K * _CH, _EMBED), jnp.float32),
        pltpu.SemaphoreType.DMA,
        pltpu.SemaphoreType.DMA,
        pltpu.SemaphoreType.DMA,
        pltpu.SemaphoreType.DMA,
    ],
    compiler_params=pltpu.CompilerParams(use_tc_tiling_on_sc=False),
)
def _emb_gather(
    table_hbm, idx_hbm, out_hbm, idx_v, buf0, buf1, buf2, gsem0, gsem1, gsem2, wsem
):
    wid = lax.axis_index("s") * _NC + lax.axis_index("c")
    pltpu.sync_copy(idx_hbm.at[pl.ds(wid * _NCH, _NCH)], idx_v)

    bufs = (buf0, buf1, buf2)
    gsems = (gsem0, gsem1, gsem2)

    def issue_group(g, buf):
        return [
            pltpu.async_copy(
                table_hbm.at[idx_v.at[g * _K + j]],
                buf.at[pl.ds(j * _CH, _CH)],
                gsems[g % 3],
            )
            for j in range(_K)
        ]

    gdesc = [None] * _NG
    wdesc = [None] * _NG
    gdesc[0] = issue_group(0, bufs[0])
    gdesc[1] = issue_group(1, bufs[1])
    for g in range(_NG):
        cur = bufs[g % 3]
        for d in gdesc[g]:
            d.wait()
        wdesc[g] = pltpu.async_copy(
            cur,
            out_hbm.at[pl.ds((wid * _NG + g) * _K * _CH, _K * _CH)],
            wsem,
        )
        if g + 2 < _NG:
            if g >= 1:
                wdesc[g - 1].wait()
            gdesc[g + 2] = issue_group(g + 2, bufs[(g + 2) % 3])
    wdesc[_NG - 2].wait()
    wdesc[_NG - 1].wait()


def kernel(inputs, embeddings):
    table_rm = _transpose_table(embeddings.T)
    idx2 = inputs.reshape(_NT, _CH).astype(jnp.int32)
    out = _emb_gather(table_rm, idx2)
    return out.reshape(inputs.shape + (_EMBED,))


# speedup vs baseline: 1.2792x; 1.2792x over previous
import functools

import jax
import jax.numpy as jnp
from jax import lax
from jax.experimental import pallas as pl
from jax.experimental.pallas import tpu as pltpu
from jax.experimental.pallas import tpu_sc as plsc

_VOCAB = 1000000
_EMBED = 32
_NC = 2
_NS = 16
_NW = _NC * _NS
_CH = 128
_K = 5
_NG = 10
_NCH = _K * _NG
_BPW = _CH * _NCH
_B = _NW * _BPW
_NT = _B // _CH

_TC_C = 32768


@functools.partial(
    pl.pallas_call,
    out_shape=jax.ShapeDtypeStruct((_VOCAB, _EMBED), jnp.float32),
    grid=((_VOCAB + _TC_C - 1) // _TC_C,),
    in_specs=[pl.BlockSpec((_EMBED, _TC_C), lambda i: (0, i))],
    out_specs=pl.BlockSpec((_TC_C, _EMBED), lambda i: (i, 0)),
)
def _transpose_table(in_ref, out_ref):
    out_ref[...] = in_ref[...].T


@functools.partial(
    pl.kernel,
    mesh=plsc.VectorSubcoreMesh(core_axis_name="c", subcore_axis_name="s"),
    out_type=jax.ShapeDtypeStruct((_B, _EMBED), jnp.float32),
    scratch_types=[
        pltpu.VMEM((_NCH, _CH), jnp.int32),
        pltpu.VMEM((_K * _CH, _EMBED), jnp.float32),
        pltpu.VMEM((_K * _CH, _EMBED), jnp.float32),
        pltpu.VMEM((_K * _CH, _EMBED), jnp.float32),
        pltpu.SemaphoreType.DMA,
        pltpu.SemaphoreType.DMA,
        pltpu.SemaphoreType.DMA,
        pltpu.SemaphoreType.DMA,
    ],
    compiler_params=pltpu.CompilerParams(use_tc_tiling_on_sc=False),
)
def _emb_gather(
    table_hbm, idx_hbm, out_hbm, idx_v, buf0, buf1, buf2, gsem0, gsem1, gsem2, wsem
):
    wid = lax.axis_index("s") * _NC + lax.axis_index("c")
    pltpu.sync_copy(idx_hbm.at[pl.ds(wid * _NCH, _NCH)], idx_v)

    bufs = (buf0, buf1, buf2)
    gsems = (gsem0, gsem1, gsem2)

    def issue_group(g, buf):
        return [
            pltpu.async_copy(
                table_hbm.at[idx_v.at[g * _K + j]],
                buf.at[pl.ds(j * _CH, _CH)],
                gsems[g % 3],
            )
            for j in range(_K)
        ]

    gdesc = [None] * _NG
    wdesc = [None] * _NG
    gdesc[0] = issue_group(0, bufs[0])
    gdesc[1] = issue_group(1, bufs[1])
    for g in range(_NG):
        cur = bufs[g % 3]
        for d in gdesc[g]:
            d.wait()
        wdesc[g] = pltpu.async_copy(
            cur,
            out_hbm.at[pl.ds((wid * _NG + g) * _K * _CH, _K * _CH)],
            wsem,
        )
        if g + 2 < _NG:
            if g >= 1:
                wdesc[g - 1].wait()
            gdesc[g + 2] = issue_group(g + 2, bufs[(g + 2) % 3])
    wdesc[_NG - 2].wait()
    wdesc[_NG - 1].wait()


def kernel(inputs, embeddings):
    table_rm = _transpose_table(embeddings.T)
    idx2 = inputs.reshape(_NT, _CH).astype(jnp.int32)
    out = _emb_gather(table_rm, idx2)
    return out.reshape(inputs.shape + (_EMBED,))


# speedup vs baseline: 1.3555x; 1.0597x over previous
import functools

import jax
import jax.numpy as jnp
from jax import lax
from jax.experimental import pallas as pl
from jax.experimental.pallas import tpu as pltpu
from jax.experimental.pallas import tpu_sc as plsc

_VOCAB = 1000000
_EMBED = 32
_NC = 2
_NS = 16
_NW = _NC * _NS
_CH = 128
_K = 5
_NG = 10
_NCH = _K * _NG
_BPW = _CH * _NCH
_B = _NW * _BPW
_NT = _B // _CH

@functools.partial(
    pl.kernel,
    mesh=plsc.VectorSubcoreMesh(core_axis_name="c", subcore_axis_name="s"),
    out_type=jax.ShapeDtypeStruct((_B, _EMBED), jnp.float32),
    scratch_types=[
        pltpu.VMEM((_NCH, _CH), jnp.int32),
        pltpu.VMEM((_K * _CH, _EMBED), jnp.float32),
        pltpu.VMEM((_K * _CH, _EMBED), jnp.float32),
        pltpu.VMEM((_K * _CH, _EMBED), jnp.float32),
        pltpu.SemaphoreType.DMA,
        pltpu.SemaphoreType.DMA,
        pltpu.SemaphoreType.DMA,
        pltpu.SemaphoreType.DMA,
    ],
    compiler_params=pltpu.CompilerParams(use_tc_tiling_on_sc=False),
)
def _emb_gather(
    table_hbm, idx_hbm, out_hbm, idx_v, buf0, buf1, buf2, gsem0, gsem1, gsem2, wsem
):
    wid = lax.axis_index("s") * _NC + lax.axis_index("c")
    pltpu.sync_copy(idx_hbm.at[pl.ds(wid * _NCH, _NCH)], idx_v)

    bufs = (buf0, buf1, buf2)
    gsems = (gsem0, gsem1, gsem2)

    def issue_group(g, buf):
        return [
            pltpu.async_copy(
                table_hbm.at[idx_v.at[g * _K + j]],
                buf.at[pl.ds(j * _CH, _CH)],
                gsems[g % 3],
            )
            for j in range(_K)
        ]

    gdesc = [None] * _NG
    wdesc = [None] * _NG
    gdesc[0] = issue_group(0, bufs[0])
    gdesc[1] = issue_group(1, bufs[1])
    for g in range(_NG):
        cur = bufs[g % 3]
        for d in gdesc[g]:
            d.wait()
        wdesc[g] = pltpu.async_copy(
            cur,
            out_hbm.at[pl.ds((wid * _NG + g) * _K * _CH, _K * _CH)],
            wsem,
        )
        if g + 2 < _NG:
            if g >= 1:
                wdesc[g - 1].wait()
            gdesc[g + 2] = issue_group(g + 2, bufs[(g + 2) % 3])
    wdesc[_NG - 2].wait()
    wdesc[_NG - 1].wait()


def kernel(inputs, embeddings):
    idx2 = inputs.reshape(_NT, _CH).astype(jnp.int32)
    out = _emb_gather(embeddings, idx2)
    return out.reshape(inputs.shape + (_EMBED,))


# speedup vs baseline: 1.6669x; 1.2297x over previous
import functools

import jax
import jax.numpy as jnp
from jax import lax
from jax.experimental import pallas as pl
from jax.experimental.pallas import tpu as pltpu
from jax.experimental.pallas import tpu_sc as plsc

_VOCAB = 1000000
_EMBED = 32
_NC = 2
_NS = 16
_NW = _NC * _NS
_CH = 128
_K = 5
_NG = 10
_NCH = _K * _NG
_BPW = _CH * _NCH
_B = _NW * _BPW
_NT = _B // _CH

@functools.partial(
    pl.kernel,
    mesh=plsc.VectorSubcoreMesh(core_axis_name="c", subcore_axis_name="s"),
    out_type=jax.ShapeDtypeStruct((_NW * _NG, _K, _CH, _EMBED), jnp.float32),
    scratch_types=[
        pltpu.VMEM((_NCH, _CH), jnp.int32),
        pltpu.VMEM((_K, _CH, _EMBED), jnp.float32),
        pltpu.VMEM((_K, _CH, _EMBED), jnp.float32),
        pltpu.VMEM((_K, _CH, _EMBED), jnp.float32),
        pltpu.SemaphoreType.DMA,
        pltpu.SemaphoreType.DMA,
        pltpu.SemaphoreType.DMA,
        pltpu.SemaphoreType.DMA,
    ],
    compiler_params=pltpu.CompilerParams(use_tc_tiling_on_sc=False),
)
def _emb_gather(
    table_hbm, idx_hbm, out_hbm, idx_v, buf0, buf1, buf2, gsem0, gsem1, gsem2, wsem
):
    wid = lax.axis_index("s") * _NC + lax.axis_index("c")
    pltpu.sync_copy(idx_hbm.at[wid], idx_v)

    bufs = (buf0, buf1, buf2)
    gsems = (gsem0, gsem1, gsem2)

    def issue_group(g, buf):
        return [
            pltpu.async_copy(
                table_hbm.at[idx_v.at[g * _K + j]],
                buf.at[j],
                gsems[g % 3],
            )
            for j in range(_K)
        ]

    gdesc = [None] * _NG
    wdesc = [None] * _NG
    gdesc[0] = issue_group(0, bufs[0])
    gdesc[1] = issue_group(1, bufs[1])
    for g in range(_NG):
        cur = bufs[g % 3]
        for d in gdesc[g]:
            d.wait()
        wdesc[g] = pltpu.async_copy(cur, out_hbm.at[wid * _NG + g], wsem)
        if g + 2 < _NG:
            if g >= 1:
                wdesc[g - 1].wait()
            gdesc[g + 2] = issue_group(g + 2, bufs[(g + 2) % 3])
    wdesc[_NG - 2].wait()
    wdesc[_NG - 1].wait()


def kernel(inputs, embeddings):
    idx3 = inputs.reshape(_NW, _NCH, _CH).astype(jnp.int32)
    out = _emb_gather(embeddings, idx3)
    return out.reshape(inputs.shape + (_EMBED,))
